# baseline (speedup 1.0000x reference)
import functools

import jax
import jax.numpy as jnp
from jax import lax
from jax.experimental import pallas as pl
from jax.experimental.pallas import tpu as pltpu
from jax.experimental.pallas import tpu_sc as plsc

NUM_EMB = 1000000
DIM = 64
BATCH = 16384
HIST = 50

NC = 2
NS = 16
NW = NC * NS
BPW = BATCH // NW
NBT = BPW // 128
NUNIT = NBT * HIST
L = 16

_mesh = plsc.VectorSubcoreMesh(core_axis_name="c", subcore_axis_name="s")


@functools.partial(
    pl.kernel,
    mesh=_mesh,
    out_type=jax.ShapeDtypeStruct((HIST, DIM, BATCH), jnp.float32),
    scratch_types=[
        pltpu.VMEM((128,), jnp.int32),
        pltpu.VMEM((128,), jnp.int32),
        pltpu.VMEM((128,), jnp.int32),
        pltpu.VMEM((128,), jnp.int32),
        pltpu.VMEM((128, 128), jnp.float32),
        pltpu.VMEM((128, 128), jnp.float32),
        pltpu.VMEM((DIM, 128), jnp.float32),
        pltpu.VMEM((DIM, 128), jnp.float32),
        pltpu.SemaphoreType.DMA,
        pltpu.SemaphoreType.DMA,
        pltpu.SemaphoreType.DMA,
        pltpu.SemaphoreType.DMA,
        pltpu.SemaphoreType.DMA,
        pltpu.SemaphoreType.DMA,
        pltpu.SemaphoreType.DMA,
        pltpu.SemaphoreType.DMA,
    ],
    compiler_params=pltpu.CompilerParams(use_tc_tiling_on_sc=True,
                                         needs_layout_passes=False),
)
def _gather_kernel(idx_hbm, table_hbm, out_hbm,
                   idx0, idx1, idx2, idx3, rows0, rows1, st0, st1,
                   si0, si1, si2, si3, sg0, sg1, sw0, sw1):
    wid = lax.axis_index("s") * NC + lax.axis_index("c")
    bw = wid * BPW

    idxv = (idx0, idx1, idx2, idx3)
    si = (si0, si1, si2, si3)
    rows = (rows0, rows1)
    stg = (st0, st1)
    sg = (sg0, sg1)
    sw = (sw0, sw1)

    def unit_hb(u):
        bt = u // HIST
        h = u - bt * HIST
        return h, bw + bt * 128

    def i_start(q, u):
        h, b0 = unit_hb(u)
        pltpu.async_copy(idx_hbm.at[h, pl.ds(b0, 128)], idxv[q], si[q])

    def i_wait(q):
        pltpu.make_async_copy(idx_hbm.at[0, pl.ds(0, 128)], idxv[q],
                              si[q]).wait()

    def g_start(b, q):
        pltpu.async_copy(table_hbm.at[idxv[q]], rows[b], sg[b])

    def g_wait(b):
        pltpu.make_async_copy(table_hbm.at[idxv[0]], rows[b], sg[b]).wait()

    def w_start(b, u):
        h, b0 = unit_hb(u)
        pltpu.async_copy(stg[b], out_hbm.at[h, :, pl.ds(b0, 128)], sw[b])

    def w_wait(b):
        pltpu.make_async_copy(stg[b], out_hbm.at[0, :, pl.ds(0, 128)],
                              sw[b]).wait()

    lanes = lax.iota(jnp.int32, L)

    def transpose(b):
        def tbody(c, carry):
            for j in range(128 // L):
                v = plsc.load_gather(rows[b], [lanes + (L * j),
                                               jnp.full((L,), c, jnp.int32)])
                stg[b][c, pl.ds(L * j, L)] = v
            return carry
        lax.fori_loop(0, DIM, tbody, 0)

    def steady(u, b, q, skip_w_wait=False):
        g_wait(b)
        if u + 4 < NUNIT:
            i_start(q, u + 4)
        if not skip_w_wait:
            w_wait(b)
        transpose(b)
        w_start(b, u)
        if u + 2 < NUNIT:
            i_wait((u + 2) % 4)
            g_start(b, (u + 2) % 4)

    i_start(0, 0)
    i_start(1, 1)
    i_start(2, 2)
    i_start(3, 3)
    i_wait(0)
    g_start(0, 0)
    i_wait(1)
    g_start(1, 1)
    steady(0, 0, 0, skip_w_wait=True)
    steady(1, 1, 1, skip_w_wait=True)
    steady(2, 0, 2)
    steady(3, 1, 3)

    def body(k, carry):
        u0 = 4 * k
        g_wait(0)
        i_start(0, u0 + 4)
        w_wait(0)
        transpose(0)
        w_start(0, u0)
        i_wait(2)
        g_start(0, 2)
        g_wait(1)
        i_start(1, u0 + 5)
        w_wait(1)
        transpose(1)
        w_start(1, u0 + 1)
        i_wait(3)
        g_start(1, 3)
        g_wait(0)
        i_start(2, u0 + 6)
        w_wait(0)
        transpose(0)
        w_start(0, u0 + 2)
        i_wait(0)
        g_start(0, 0)
        g_wait(1)
        i_start(3, u0 + 7)
        w_wait(1)
        transpose(1)
        w_start(1, u0 + 3)
        i_wait(1)
        g_start(1, 1)
        return carry

    lax.fori_loop(1, NUNIT // 4 - 1, body, 0)

    steady(NUNIT - 4, 0, 0)
    steady(NUNIT - 3, 1, 1)
    steady(NUNIT - 2, 0, 2)
    steady(NUNIT - 1, 1, 3)
    w_wait(0)
    w_wait(1)


def kernel(input_, weight):
    idx2 = input_.T
    wpad = jnp.pad(weight, ((0, 0), (0, 128 - DIM)))
    out_t = _gather_kernel(idx2, wpad)
    return jnp.transpose(out_t, (2, 0, 1))


# speedup vs baseline: 1.1481x; 1.1481x over previous
import functools

import jax
import jax.numpy as jnp
from jax import lax
from jax.experimental import pallas as pl
from jax.experimental.pallas import tpu as pltpu
from jax.experimental.pallas import tpu_sc as plsc

NUM_EMB = 1000000
DIM = 64
BATCH = 16384
HIST = 50

NC = 2
NS = 16
NW = NC * NS
BPW = BATCH // NW
NBT = BPW // 128
NUNIT = NBT * HIST
L = 16

_mesh = plsc.VectorSubcoreMesh(core_axis_name="c", subcore_axis_name="s")


@functools.partial(
    pl.kernel,
    mesh=_mesh,
    out_type=jax.ShapeDtypeStruct((HIST, DIM, BATCH), jnp.float32),
    scratch_types=[
        pltpu.VMEM((128,), jnp.int32),
        pltpu.VMEM((128,), jnp.int32),
        pltpu.VMEM((128,), jnp.int32),
        pltpu.VMEM((128,), jnp.int32),
        pltpu.VMEM((128, 128), jnp.float32),
        pltpu.VMEM((128, 128), jnp.float32),
        pltpu.VMEM((DIM, 128), jnp.float32),
        pltpu.VMEM((DIM, 128), jnp.float32),
        pltpu.SemaphoreType.DMA,
        pltpu.SemaphoreType.DMA,
        pltpu.SemaphoreType.DMA,
        pltpu.SemaphoreType.DMA,
        pltpu.SemaphoreType.DMA,
        pltpu.SemaphoreType.DMA,
        pltpu.SemaphoreType.DMA,
        pltpu.SemaphoreType.DMA,
    ],
    compiler_params=pltpu.CompilerParams(use_tc_tiling_on_sc=True,
                                         needs_layout_passes=False),
)
def _gather_kernel(idx_hbm, table_hbm, out_hbm,
                   idx0, idx1, idx2, idx3, rows0, rows1, st0, st1,
                   si0, si1, si2, si3, sg0, sg1, sw0, sw1):
    wid = lax.axis_index("s") * NC + lax.axis_index("c")
    bw = wid * BPW

    idxv = (idx0, idx1, idx2, idx3)
    si = (si0, si1, si2, si3)
    rows = (rows0, rows1)
    stg = (st0, st1)
    sg = (sg0, sg1)
    sw = (sw0, sw1)

    def unit_hb(u):
        bt = u // HIST
        h = u - bt * HIST
        return h, bw + bt * 128

    def i_start(q, u):
        h, b0 = unit_hb(u)
        pltpu.async_copy(idx_hbm.at[h, pl.ds(b0, 128)], idxv[q], si[q])

    def i_wait(q):
        pltpu.make_async_copy(idx_hbm.at[0, pl.ds(0, 128)], idxv[q],
                              si[q]).wait()

    def g_start(b, q):
        pltpu.async_copy(table_hbm.at[idxv[q]], rows[b], sg[b])

    def g_wait(b):
        pltpu.make_async_copy(table_hbm.at[idxv[0]], rows[b], sg[b]).wait()

    def w_start(b, u):
        h, b0 = unit_hb(u)
        pltpu.async_copy(stg[b], out_hbm.at[h, :, pl.ds(b0, 128)], sw[b])

    def w_wait(b):
        pltpu.make_async_copy(stg[b], out_hbm.at[0, :, pl.ds(0, 128)],
                              sw[b]).wait()

    lanes = lax.iota(jnp.int32, L)
    rowidx = tuple(lanes + (L * j) for j in range(128 // L))

    def transpose(b):
        def tbody(c8, carry):
            c0 = c8 * 8
            for ci in range(8):
                c = c0 + ci
                col = jnp.full((L,), c, jnp.int32)
                vs = [plsc.load_gather(rows[b], [rowidx[j], col])
                      for j in range(128 // L)]
                for j in range(128 // L):
                    stg[b][c, pl.ds(L * j, L)] = vs[j]
            return carry
        lax.fori_loop(0, DIM // 8, tbody, 0)

    def steady(u, b, q, skip_w_wait=False):
        g_wait(b)
        if u + 4 < NUNIT:
            i_start(q, u + 4)
        if not skip_w_wait:
            w_wait(b)
        transpose(b)
        w_start(b, u)
        if u + 2 < NUNIT:
            i_wait((u + 2) % 4)
            g_start(b, (u + 2) % 4)

    i_start(0, 0)
    i_start(1, 1)
    i_start(2, 2)
    i_start(3, 3)
    i_wait(0)
    g_start(0, 0)
    i_wait(1)
    g_start(1, 1)
    steady(0, 0, 0, skip_w_wait=True)
    steady(1, 1, 1, skip_w_wait=True)
    steady(2, 0, 2)
    steady(3, 1, 3)

    def body(k, carry):
        u0 = 4 * k
        g_wait(0)
        i_start(0, u0 + 4)
        w_wait(0)
        transpose(0)
        w_start(0, u0)
        i_wait(2)
        g_start(0, 2)
        g_wait(1)
        i_start(1, u0 + 5)
        w_wait(1)
        transpose(1)
        w_start(1, u0 + 1)
        i_wait(3)
        g_start(1, 3)
        g_wait(0)
        i_start(2, u0 + 6)
        w_wait(0)
        transpose(0)
        w_start(0, u0 + 2)
        i_wait(0)
        g_start(0, 0)
        g_wait(1)
        i_start(3, u0 + 7)
        w_wait(1)
        transpose(1)
        w_start(1, u0 + 3)
        i_wait(1)
        g_start(1, 1)
        return carry

    lax.fori_loop(1, NUNIT // 4 - 1, body, 0)

    steady(NUNIT - 4, 0, 0)
    steady(NUNIT - 3, 1, 1)
    steady(NUNIT - 2, 0, 2)
    steady(NUNIT - 1, 1, 3)
    w_wait(0)
    w_wait(1)


def kernel(input_, weight):
    idx2 = input_.T
    wpad = jnp.pad(weight, ((0, 0), (0, 128 - DIM)))
    out_t = _gather_kernel(idx2, wpad)
    return jnp.transpose(out_t, (2, 0, 1))


# speedup vs baseline: 2.2913x; 1.9958x over previous
import functools

import jax
import jax.numpy as jnp
from jax import lax
from jax.experimental import pallas as pl
from jax.experimental.pallas import tpu as pltpu
from jax.experimental.pallas import tpu_sc as plsc

NUM_EMB = 1000000
DIM = 64
BATCH = 16384
HIST = 50

NC = 2
NS = 16
NW = NC * NS
BPW = BATCH // NW
NBT = BPW // 128
NUNIT = NBT * HIST
L = 16

_mesh = plsc.VectorSubcoreMesh(core_axis_name="c", subcore_axis_name="s")


@functools.partial(
    pl.kernel,
    mesh=_mesh,
    out_type=jax.ShapeDtypeStruct((HIST, DIM, BATCH), jnp.float32),
    scratch_types=[
        pltpu.VMEM((128,), jnp.int32),
        pltpu.VMEM((128,), jnp.int32),
        pltpu.VMEM((128,), jnp.int32),
        pltpu.VMEM((128,), jnp.int32),
        pltpu.VMEM((128, 128), jnp.float32),
        pltpu.VMEM((128, 128), jnp.float32),
        pltpu.VMEM((DIM, 128), jnp.float32),
        pltpu.VMEM((DIM, 128), jnp.float32),
        pltpu.SemaphoreType.DMA,
        pltpu.SemaphoreType.DMA,
        pltpu.SemaphoreType.DMA,
        pltpu.SemaphoreType.DMA,
        pltpu.SemaphoreType.DMA,
        pltpu.SemaphoreType.DMA,
        pltpu.SemaphoreType.DMA,
        pltpu.SemaphoreType.DMA,
    ],
    compiler_params=pltpu.CompilerParams(use_tc_tiling_on_sc=True,
                                         needs_layout_passes=False),
)
def _gather_kernel(idx_hbm, table_hbm, out_hbm,
                   idx0, idx1, idx2, idx3, rows0, rows1, st0, st1,
                   si0, si1, si2, si3, sg0, sg1, sw0, sw1):
    wid = lax.axis_index("s") * NC + lax.axis_index("c")
    bw = wid * BPW

    idxv = (idx0, idx1, idx2, idx3)
    si = (si0, si1, si2, si3)
    rows = (rows0, rows1)
    stg = (st0, st1)
    sg = (sg0, sg1)
    sw = (sw0, sw1)

    def unit_hb(u):
        bt = u // HIST
        h = u - bt * HIST
        return h, bw + bt * 128

    def i_start(q, u):
        h, b0 = unit_hb(u)
        pltpu.async_copy(idx_hbm.at[h, pl.ds(b0, 128)], idxv[q], si[q])

    def i_wait(q):
        pltpu.make_async_copy(idx_hbm.at[0, pl.ds(0, 128)], idxv[q],
                              si[q]).wait()

    def g_start(b, q):
        pltpu.async_copy(table_hbm.at[idxv[q]], rows[b], sg[b])

    def g_wait(b):
        pltpu.make_async_copy(table_hbm.at[idxv[0]], rows[b], sg[b]).wait()

    def w_start(b, u):
        h, b0 = unit_hb(u)
        pltpu.async_copy(stg[b], out_hbm.at[h, :, pl.ds(b0, 128)], sw[b])

    def w_wait(b):
        pltpu.make_async_copy(stg[b], out_hbm.at[0, :, pl.ds(0, 128)],
                              sw[b]).wait()

    lanes = lax.iota(jnp.int32, L)
    rowidx = tuple(lanes + (L * j) for j in range(128 // L))

    def transpose(b):
        @functools.partial(plsc.parallel_loop, 0, DIM, unroll=4)
        def tbody(c):
            col = jnp.full((L,), c, jnp.int32)
            vs = [plsc.load_gather(rows[b], [rowidx[j], col])
                  for j in range(128 // L)]
            for j in range(128 // L):
                stg[b][c, pl.ds(L * j, L)] = vs[j]

    def steady(u, b, q, skip_w_wait=False):
        g_wait(b)
        if u + 4 < NUNIT:
            i_start(q, u + 4)
        if not skip_w_wait:
            w_wait(b)
        transpose(b)
        w_start(b, u)
        if u + 2 < NUNIT:
            i_wait((u + 2) % 4)
            g_start(b, (u + 2) % 4)

    i_start(0, 0)
    i_start(1, 1)
    i_start(2, 2)
    i_start(3, 3)
    i_wait(0)
    g_start(0, 0)
    i_wait(1)
    g_start(1, 1)
    steady(0, 0, 0, skip_w_wait=True)
    steady(1, 1, 1, skip_w_wait=True)
    steady(2, 0, 2)
    steady(3, 1, 3)

    def body(k, carry):
        u0 = 4 * k
        g_wait(0)
        i_start(0, u0 + 4)
        w_wait(0)
        transpose(0)
        w_start(0, u0)
        i_wait(2)
        g_start(0, 2)
        g_wait(1)
        i_start(1, u0 + 5)
        w_wait(1)
        transpose(1)
        w_start(1, u0 + 1)
        i_wait(3)
        g_start(1, 3)
        g_wait(0)
        i_start(2, u0 + 6)
        w_wait(0)
        transpose(0)
        w_start(0, u0 + 2)
        i_wait(0)
        g_start(0, 0)
        g_wait(1)
        i_start(3, u0 + 7)
        w_wait(1)
        transpose(1)
        w_start(1, u0 + 3)
        i_wait(1)
        g_start(1, 1)
        return carry

    lax.fori_loop(1, NUNIT // 4 - 1, body, 0)

    steady(NUNIT - 4, 0, 0)
    steady(NUNIT - 3, 1, 1)
    steady(NUNIT - 2, 0, 2)
    steady(NUNIT - 1, 1, 3)
    w_wait(0)
    w_wait(1)


def kernel(input_, weight):
    idx2 = input_.T
    wpad = jnp.pad(weight, ((0, 0), (0, 128 - DIM)))
    out_t = _gather_kernel(idx2, wpad)
    return jnp.transpose(out_t, (2, 0, 1))


# speedup vs baseline: 3.3296x; 1.4532x over previous
import functools

import jax
import jax.numpy as jnp
from jax import lax
from jax.experimental import pallas as pl
from jax.experimental.pallas import tpu as pltpu
from jax.experimental.pallas import tpu_sc as plsc

NUM_EMB = 1000000
DIM = 64
BATCH = 16384
HIST = 50

NC = 2
NS = 16
NW = NC * NS
BPW = BATCH // NW
NBT = BPW // 128
NUNIT = NBT * HIST
L = 16

_mesh = plsc.VectorSubcoreMesh(core_axis_name="c", subcore_axis_name="s")


@functools.partial(
    pl.kernel,
    mesh=_mesh,
    out_type=jax.ShapeDtypeStruct((HIST, DIM, BATCH), jnp.float32),
    scratch_types=[
        pltpu.VMEM((128,), jnp.int32),
        pltpu.VMEM((128,), jnp.int32),
        pltpu.VMEM((128,), jnp.int32),
        pltpu.VMEM((128,), jnp.int32),
        pltpu.VMEM((128, 128), jnp.float32),
        pltpu.VMEM((128, 128), jnp.float32),
        pltpu.VMEM((DIM, 128), jnp.float32),
        pltpu.VMEM((DIM, 128), jnp.float32),
        pltpu.SemaphoreType.DMA,
        pltpu.SemaphoreType.DMA,
        pltpu.SemaphoreType.DMA,
        pltpu.SemaphoreType.DMA,
        pltpu.SemaphoreType.DMA,
        pltpu.SemaphoreType.DMA,
        pltpu.SemaphoreType.DMA,
        pltpu.SemaphoreType.DMA,
    ],
    compiler_params=pltpu.CompilerParams(use_tc_tiling_on_sc=True,
                                         needs_layout_passes=False),
)
def _gather_kernel(idx_hbm, table_hbm, out_hbm,
                   idx0, idx1, idx2, idx3, rows0, rows1, st0, st1,
                   si0, si1, si2, si3, sg0, sg1, sw0, sw1):
    wid = lax.axis_index("s") * NC + lax.axis_index("c")
    bw = wid * BPW

    idxv = (idx0, idx1, idx2, idx3)
    si = (si0, si1, si2, si3)
    rows = (rows0, rows1)
    stg = (st0, st1)
    sg = (sg0, sg1)
    sw = (sw0, sw1)

    def unit_hb(u):
        bt = u // HIST
        h = u - bt * HIST
        return h, bw + bt * 128

    def i_start(q, u):
        h, b0 = unit_hb(u)
        pltpu.async_copy(idx_hbm.at[h, pl.ds(b0, 128)], idxv[q], si[q])

    def i_wait(q):
        pltpu.make_async_copy(idx_hbm.at[0, pl.ds(0, 128)], idxv[q],
                              si[q]).wait()

    def g_start(b, q):
        pltpu.async_copy(table_hbm.at[idxv[q]], rows[b], sg[b])

    def g_wait(b):
        pltpu.make_async_copy(table_hbm.at[idxv[0]], rows[b], sg[b]).wait()

    def w_start(b, u):
        h, b0 = unit_hb(u)
        pltpu.async_copy(stg[b], out_hbm.at[h, :, pl.ds(b0, 128)], sw[b])

    def w_wait(b):
        pltpu.make_async_copy(stg[b], out_hbm.at[0, :, pl.ds(0, 128)],
                              sw[b]).wait()

    lanes = lax.iota(jnp.int32, L)
    rowidx = tuple(lanes + (L * j) for j in range(128 // L))

    def transpose(b):
        @functools.partial(plsc.parallel_loop, 0, DIM, unroll=4)
        def tbody(c):
            col = jnp.full((L,), c, jnp.int32)
            vs = [plsc.load_gather(rows[b], [rowidx[j], col])
                  for j in range(128 // L)]
            for j in range(128 // L):
                stg[b][c, pl.ds(L * j, L)] = vs[j]

    def steady(u, b, q, skip_w_wait=False):
        g_wait(b)
        if u + 4 < NUNIT:
            i_start(q, u + 4)
        if not skip_w_wait:
            w_wait(b)
        transpose(b)
        w_start(b, u)
        if u + 2 < NUNIT:
            i_wait((u + 2) % 4)
            g_start(b, (u + 2) % 4)

    i_start(0, 0)
    i_start(1, 1)
    i_start(2, 2)
    i_start(3, 3)
    i_wait(0)
    g_start(0, 0)
    i_wait(1)
    g_start(1, 1)
    steady(0, 0, 0, skip_w_wait=True)
    steady(1, 1, 1, skip_w_wait=True)
    steady(2, 0, 2)
    steady(3, 1, 3)

    def body(k, carry):
        u0 = 4 * k
        g_wait(0)
        i_start(0, u0 + 4)
        w_wait(0)
        transpose(0)
        w_start(0, u0)
        i_wait(2)
        g_start(0, 2)
        g_wait(1)
        i_start(1, u0 + 5)
        w_wait(1)
        transpose(1)
        w_start(1, u0 + 1)
        i_wait(3)
        g_start(1, 3)
        g_wait(0)
        i_start(2, u0 + 6)
        w_wait(0)
        transpose(0)
        w_start(0, u0 + 2)
        i_wait(0)
        g_start(0, 0)
        g_wait(1)
        i_start(3, u0 + 7)
        w_wait(1)
        transpose(1)
        w_start(1, u0 + 3)
        i_wait(1)
        g_start(1, 1)
        return carry

    lax.fori_loop(1, NUNIT // 4 - 1, body, 0)

    steady(NUNIT - 4, 0, 0)
    steady(NUNIT - 3, 1, 1)
    steady(NUNIT - 2, 0, 2)
    steady(NUNIT - 1, 1, 3)
    w_wait(0)
    w_wait(1)


NTC = NUM_EMB // 128
REM = NUM_EMB - NTC * 128
CUNIT = 246


@functools.partial(
    pl.kernel,
    mesh=_mesh,
    out_type=jax.ShapeDtypeStruct((NUM_EMB, 128), jnp.float32),
    scratch_types=[
        pltpu.VMEM((DIM, 128), jnp.float32),
        pltpu.VMEM((DIM, 128), jnp.float32),
        pltpu.VMEM((128, 128), jnp.float32),
        pltpu.VMEM((128, 128), jnp.float32),
        pltpu.SemaphoreType.DMA,
        pltpu.SemaphoreType.DMA,
        pltpu.SemaphoreType.DMA,
        pltpu.SemaphoreType.DMA,
    ],
    compiler_params=pltpu.CompilerParams(use_tc_tiling_on_sc=True,
                                         needs_layout_passes=False),
)
def _convert_kernel(wt_hbm, tail_hbm, out_hbm, src0, src1, dst0, dst1,
                    sr0, sr1, sw0, sw1):
    wid = lax.axis_index("s") * NC + lax.axis_index("c")

    src = (src0, src1)
    dst = (dst0, dst1)
    sr = (sr0, sr1)
    sw = (sw0, sw1)

    def tc_of(n):
        tc = wid + NW * n
        return jnp.where(tc < NTC, tc, 0)

    def r_start(b, n):
        r0 = pl.multiple_of(tc_of(n) * 128, 128)
        pltpu.async_copy(wt_hbm.at[:, pl.ds(r0, 128)], src[b], sr[b])

    def r_wait(b):
        pltpu.make_async_copy(wt_hbm.at[:, pl.ds(0, 128)], src[b],
                              sr[b]).wait()

    def w_start(b, n):
        r0 = pl.multiple_of(tc_of(n) * 128, 128)
        pltpu.async_copy(dst[b], out_hbm.at[pl.ds(r0, 128)], sw[b])

    def w_wait(b):
        pltpu.make_async_copy(dst[b], out_hbm.at[pl.ds(0, 128)], sw[b]).wait()

    lanes = lax.iota(jnp.int32, L)
    cidx = tuple(lanes + (L * j) for j in range(DIM // L))

    def transpose(b, nrows):
        @functools.partial(plsc.parallel_loop, 0, nrows, unroll=4)
        def tbody(r):
            col = jnp.full((L,), r, jnp.int32)
            vs = [plsc.load_gather(src[b], [cidx[j], col])
                  for j in range(DIM // L)]
            for j in range(DIM // L):
                dst[b][r, pl.ds(L * j, L)] = vs[j]

    def steady(n, b, skip_w_wait=False, do_start=True):
        r_wait(b)
        if not skip_w_wait:
            w_wait(b)
        transpose(b, 128)
        w_start(b, n)
        if do_start:
            r_start(b, n + 2)

    r_start(0, 0)
    r_start(1, 1)
    steady(0, 0, skip_w_wait=True)
    steady(1, 1, skip_w_wait=True)

    def body(k, carry):
        steady(2 * k, 0)
        steady(2 * k + 1, 1)
        return carry

    lax.fori_loop(1, CUNIT // 2 - 1, body, 0)

    steady(CUNIT - 2, 0, do_start=False)
    steady(CUNIT - 1, 1, do_start=False)
    w_wait(0)
    w_wait(1)

    @pl.when(wid == 0)
    def _partial():
        pltpu.sync_copy(tail_hbm, src0.at[pl.ds(0, DIM)])
        pltpu.sync_copy(src0.at[pl.ds(0, DIM)],
                        out_hbm.at[pl.ds(NTC * 128, REM)])


def kernel(input_, weight):
    idx2 = input_.T
    tail = jnp.pad(weight[NTC * 128:], ((0, 0), (0, 128 - DIM)))
    wpad = _convert_kernel(weight.T, tail)
    out_t = _gather_kernel(idx2, wpad)
    return jnp.transpose(out_t, (2, 0, 1))
